# baseline (speedup 1.0000x reference)
import functools
import math

import jax
import jax.numpy as jnp
from jax import lax
from jax.experimental import pallas as pl
from jax.experimental.pallas import tpu as pltpu
from jax.experimental.pallas import tpu_sc as plsc

D_MODEL_K = 64
SCALE_K = math.sqrt(D_MODEL_K)

NC = 2
NS = 16
NW = NC * NS
LANES = 128
NBUF = 4


def _emb_body(x_hbm, lut_hbm, out_hbm, idx_v,
              gb0, gb1, gb2, gb3, sb0, sb1, sb2, sb3,
              gs0, gs1, gs2, gs3, ss0, ss1, ss2, ss3):
    gbufs = (gb0, gb1, gb2, gb3)
    sbufs = (sb0, sb1, sb2, sb3)
    gsems = (gs0, gs1, gs2, gs3)
    ssems = (ss0, ss1, ss2, ss3)

    wid = lax.axis_index("s") * NC + lax.axis_index("c")
    n_seq = x_hbm.shape[0]

    pltpu.sync_copy(x_hbm.at[:, wid], idx_v)

    for b in range(NBUF):
        pltpu.async_copy(lut_hbm.at[idx_v.at[b]], gbufs[b], gsems[b])

    iotas = [jnp.arange(16, dtype=jnp.int32) + (16 * u) for u in range(8)]

    def outer_body(outer, carry):
        for b in range(NBUF):
            g = outer * NBUF + b
            pltpu.make_async_copy(
                lut_hbm.at[idx_v.at[b]], gbufs[b], gsems[b]).wait()

            def tr_k(k, c2, gb=gbufs[b], sb=sbufs[b]):
                for d8 in range(8):
                    col = jnp.broadcast_to(k * 8 + d8, (16,)).astype(jnp.int32)
                    for u in range(8):
                        v = plsc.load_gather(gb, [iotas[u], col])
                        sb[k, d8, pl.ds(16 * u, 16)] = v * SCALE_K
                return c2

            lax.fori_loop(0, 8, tr_k, 0)

            @pl.when(outer < (n_seq // NBUF) - 1)
            def _issue(b=b, g=g):
                pltpu.async_copy(
                    lut_hbm.at[idx_v.at[g + NBUF]], gbufs[b], gsems[b])

            @pl.when(outer > 0)
            def _drain(b=b, g=g):
                pltpu.make_async_copy(
                    sbufs[b], out_hbm.at[g, :, wid], ssems[b]).wait()

            pltpu.async_copy(sbufs[b], out_hbm.at[g, :, wid], ssems[b])
        return carry

    lax.fori_loop(0, n_seq // NBUF, outer_body, 0)

    for b in range(NBUF):
        g = n_seq - NBUF + b
        pltpu.make_async_copy(
            sbufs[b], out_hbm.at[g, :, wid], ssems[b]).wait()


@jax.jit
def _emb_call(x_cols, lut):
    n_seq = x_cols.shape[0]
    mesh = plsc.VectorSubcoreMesh(core_axis_name="c", subcore_axis_name="s")
    fn = functools.partial(
        pl.kernel,
        out_type=jax.ShapeDtypeStruct((n_seq, 8, NW, 8, LANES), jnp.float32),
        mesh=mesh,
        scratch_types=(
            [pltpu.VMEM((n_seq, LANES), jnp.int32)]
            + [pltpu.VMEM((LANES, D_MODEL_K), jnp.float32)] * NBUF
            + [pltpu.VMEM((8, 8, LANES), jnp.float32)] * NBUF
            + [pltpu.SemaphoreType.DMA] * (2 * NBUF)
        ),
        compiler_params=pltpu.CompilerParams(
            use_tc_tiling_on_sc=False, needs_layout_passes=False),
    )(_emb_body)
    return fn(x_cols, lut)


def kernel(x, lut):
    nb, ns = x.shape
    x_cols = jnp.transpose(x, (1, 0)).reshape(ns, NW, LANES).astype(jnp.int32)
    o5 = _emb_call(x_cols, lut)
    out = jnp.transpose(o5, (2, 4, 0, 1, 3)).reshape(nb, ns, D_MODEL_K)
    return out


# speedup vs baseline: 1.5190x; 1.5190x over previous
import functools
import math

import jax
import jax.numpy as jnp
from jax import lax
from jax.experimental import pallas as pl
from jax.experimental.pallas import tpu as pltpu
from jax.experimental.pallas import tpu_sc as plsc

D_MODEL_K = 64
SCALE_K = math.sqrt(D_MODEL_K)

NC = 2
NS = 16
NW = NC * NS
LANES = 128
NBUF = 4


def _emb_body(x_hbm, lut_hbm, out_hbm, idx_v,
              gb0, gb1, gb2, gb3, sb0, sb1, sb2, sb3,
              gs0, gs1, gs2, gs3, ss0, ss1, ss2, ss3):
    gbufs = (gb0, gb1, gb2, gb3)
    sbufs = (sb0, sb1, sb2, sb3)
    gsems = (gs0, gs1, gs2, gs3)
    ssems = (ss0, ss1, ss2, ss3)

    wid = lax.axis_index("s") * NC + lax.axis_index("c")
    n_seq = x_hbm.shape[0]

    pltpu.sync_copy(x_hbm.at[:, wid], idx_v)

    for b in range(NBUF):
        pltpu.async_copy(lut_hbm.at[idx_v.at[b]], gbufs[b], gsems[b])

    iota = lax.iota(jnp.int32, 16)
    kvecs = [iota // 8 + 2 * q for q in range(4)]
    d8vec = iota % 8

    def outer_body(outer, carry):
        for b in range(NBUF):
            g = outer * NBUF + b
            pltpu.make_async_copy(
                lut_hbm.at[idx_v.at[b]], gbufs[b], gsems[b]).wait()

            def tr_t(t, gb=gbufs[b], sb=sbufs[b]):
                tv = jnp.broadcast_to(t, (16,)).astype(jnp.int32)
                for q in range(4):
                    v = gb[t, pl.ds(16 * q, 16)] * SCALE_K
                    plsc.store_scatter(sb, [kvecs[q], d8vec, tv], v)

            plsc.parallel_loop(0, LANES, unroll=4)(tr_t)

            @pl.when(outer < (n_seq // NBUF) - 1)
            def _issue(b=b, g=g):
                pltpu.async_copy(
                    lut_hbm.at[idx_v.at[g + NBUF]], gbufs[b], gsems[b])

            @pl.when(outer > 0)
            def _drain(b=b, g=g):
                pltpu.make_async_copy(
                    sbufs[b], out_hbm.at[g, :, wid], ssems[b]).wait()

            pltpu.async_copy(sbufs[b], out_hbm.at[g, :, wid], ssems[b])
        return carry

    lax.fori_loop(0, n_seq // NBUF, outer_body, 0)

    for b in range(NBUF):
        g = n_seq - NBUF + b
        pltpu.make_async_copy(
            sbufs[b], out_hbm.at[g, :, wid], ssems[b]).wait()


@jax.jit
def _emb_call(x_cols, lut):
    n_seq = x_cols.shape[0]
    mesh = plsc.VectorSubcoreMesh(core_axis_name="c", subcore_axis_name="s")
    fn = functools.partial(
        pl.kernel,
        out_type=jax.ShapeDtypeStruct((n_seq, 8, NW, 8, LANES), jnp.float32),
        mesh=mesh,
        scratch_types=(
            [pltpu.VMEM((n_seq, LANES), jnp.int32)]
            + [pltpu.VMEM((LANES, D_MODEL_K), jnp.float32)] * NBUF
            + [pltpu.VMEM((8, 8, LANES), jnp.float32)] * NBUF
            + [pltpu.SemaphoreType.DMA] * (2 * NBUF)
        ),
        compiler_params=pltpu.CompilerParams(
            use_tc_tiling_on_sc=False, needs_layout_passes=False),
    )(_emb_body)
    return fn(x_cols, lut)


def kernel(x, lut):
    nb, ns = x.shape
    x_cols = jnp.transpose(x, (1, 0)).reshape(ns, NW, LANES).astype(jnp.int32)
    o5 = _emb_call(x_cols, lut)
    out = jnp.transpose(o5, (2, 4, 0, 1, 3)).reshape(nb, ns, D_MODEL_K)
    return out


# speedup vs baseline: 1.6127x; 1.0617x over previous
import functools
import math

import jax
import jax.numpy as jnp
from jax import lax
from jax.experimental import pallas as pl
from jax.experimental.pallas import tpu as pltpu
from jax.experimental.pallas import tpu_sc as plsc

D_MODEL = 64
SCALE = math.sqrt(D_MODEL)

NC = 2
NS = 16
NW = NC * NS
CHUNK = 128
NBUF = 4


def _emb_body(x_hbm, lut_hbm, out_hbm, idx_v,
              gb0, gb1, gb2, gb3, sb0, sb1, sb2, sb3,
              gs0, gs1, gs2, gs3, ss0, ss1, ss2, ss3):
    gbufs = (gb0, gb1, gb2, gb3)
    sbufs = (sb0, sb1, sb2, sb3)
    gsems = (gs0, gs1, gs2, gs3)
    ssems = (ss0, ss1, ss2, ss3)

    n_chunks = idx_v.shape[0]
    wid = lax.axis_index("s") * NC + lax.axis_index("c")
    chunk0 = wid * n_chunks

    pltpu.sync_copy(x_hbm.at[pl.ds(chunk0, n_chunks)], idx_v)

    for b in range(NBUF):
        pltpu.async_copy(lut_hbm.at[idx_v.at[b]], gbufs[b], gsems[b])

    def outer_body(outer, carry):
        for b in range(NBUF):
            j = outer * NBUF + b
            pltpu.make_async_copy(
                lut_hbm.at[idx_v.at[j]], gbufs[b], gsems[b]).wait()

            @pl.when(outer > 0)
            def _drain(b=b, j=j):
                pltpu.make_async_copy(
                    sbufs[b],
                    out_hbm.at[pl.ds(CHUNK * (chunk0 + j - NBUF), CHUNK)],
                    ssems[b]).wait()

            def sc_t(t, gb=gbufs[b], sb=sbufs[b]):
                for q in range(4):
                    sb[t, pl.ds(16 * q, 16)] = (
                        gb[t, pl.ds(16 * q, 16)] * SCALE)

            plsc.parallel_loop(0, CHUNK, unroll=4)(sc_t)

            @pl.when(j + NBUF < n_chunks)
            def _issue(b=b, j=j):
                pltpu.async_copy(
                    lut_hbm.at[idx_v.at[j + NBUF]], gbufs[b], gsems[b])

            pltpu.async_copy(
                sbufs[b],
                out_hbm.at[pl.ds(CHUNK * (chunk0 + j), CHUNK)],
                ssems[b])
        return carry

    lax.fori_loop(0, n_chunks // NBUF, outer_body, 0)

    for b in range(NBUF):
        j = n_chunks - NBUF + b
        pltpu.make_async_copy(
            sbufs[b],
            out_hbm.at[pl.ds(CHUNK * (chunk0 + j), CHUNK)],
            ssems[b]).wait()


@jax.jit
def _emb_call(x2, lut):
    n_rows = x2.shape[0]
    chunks_w = n_rows // NW
    mesh = plsc.VectorSubcoreMesh(core_axis_name="c", subcore_axis_name="s")
    fn = functools.partial(
        pl.kernel,
        out_type=jax.ShapeDtypeStruct((n_rows * CHUNK, D_MODEL), jnp.float32),
        mesh=mesh,
        scratch_types=(
            [pltpu.VMEM((chunks_w, CHUNK), jnp.int32)]
            + [pltpu.VMEM((CHUNK, D_MODEL), jnp.float32)] * (2 * NBUF)
            + [pltpu.SemaphoreType.DMA] * (2 * NBUF)
        ),
        compiler_params=pltpu.CompilerParams(
            use_tc_tiling_on_sc=False, needs_layout_passes=False),
    )(_emb_body)
    return fn(x2, lut)


def kernel(x, lut):
    nb, ns = x.shape
    x2 = x.reshape(nb * ns // CHUNK, CHUNK).astype(jnp.int32)
    out = _emb_call(x2, lut)
    return out.reshape(nb, ns, D_MODEL)
